# Initial kernel scaffold; baseline (speedup 1.0000x reference)
import jax
import jax.numpy as jnp
from jax.experimental import pallas as pl


def kernel(x, adjs, edgenum, fc1_w, fc1_b, fc2_w, fc2_b, fc3_w, fc3_b, fc4_w, fc4_b, g0_wself, g0_wrel, g0_b, g1_wself, g1_wrel, g1_b):
    raise NotImplementedError("write your pallas kernel here")


# speedup vs baseline: 9.6397x; 9.6397x over previous
import jax
import jax.numpy as jnp
from jax.experimental import pallas as pl
from jax.experimental.pallas import tpu as pltpu

_S, _N, _F, _H, _R = 8, 1024, 128, 256, 4


def _leaky(v):
    return jnp.where(v >= 0, v, 0.01 * v)


def _rgcn_last_step_kernel(
    x_ref, adj_ref,
    fc1_w_ref, fc1_b_ref, fc2_w_ref, fc2_b_ref,
    fc3_w_ref, fc3_b_ref, fc4_w_ref, fc4_b_ref,
    g0_ws_ref, g0_wr_ref, g0_b_ref,
    g1_ws_ref, g1_wr_ref, g1_b_ref,
    out_ref,
):
    f32 = jnp.float32
    x = x_ref[0, 0]
    h = _leaky(jnp.dot(x, fc1_w_ref[...], preferred_element_type=f32)
               + fc1_b_ref[...])
    h = _leaky(jnp.dot(h, fc2_w_ref[...], preferred_element_type=f32)
               + fc2_b_ref[...])

    def rgcn(h, ws_ref, wr_ref, b_ref):
        acc = jnp.dot(h, ws_ref[...], preferred_element_type=f32) + b_ref[...]
        for r in range(_R):
            adj = adj_ref[0, 0, r]
            deg = jnp.sum(adj, axis=1, keepdims=True) + 1e-6
            agg = jnp.dot(adj, h, preferred_element_type=f32) / deg
            acc = acc + jnp.dot(agg, wr_ref[r], preferred_element_type=f32)
        return _leaky(acc)

    h = rgcn(h, g0_ws_ref, g0_wr_ref, g0_b_ref)
    h = rgcn(h, g1_ws_ref, g1_wr_ref, g1_b_ref)

    o = _leaky(jnp.dot(h, fc3_w_ref[...], preferred_element_type=f32)
               + fc3_b_ref[...])
    y = jnp.sum(o * fc4_w_ref[...], axis=1, keepdims=True) + fc4_b_ref[0, 0]
    out_ref[0] = y


def kernel(x, adjs, edgenum, fc1_w, fc1_b, fc2_w, fc2_b, fc3_w, fc3_b,
           fc4_w, fc4_b, g0_wself, g0_wrel, g0_b, g1_wself, g1_wrel, g1_b):
    del edgenum
    last = _S - 1

    def full(shape):
        return pl.BlockSpec(shape, lambda i: tuple(0 for _ in shape))

    in_specs = [
        pl.BlockSpec((1, 1, _N, _F), lambda i: (0, last, 0, 0)),
        pl.BlockSpec((1, 1, _R, _N, _N), lambda i: (0, last, 0, 0, 0)),
        full((_F, _H)), full((1, _H)),
        full((_H, _H)), full((1, _H)),
        full((_H, _H)), full((1, _H)),
        full((1, _H)), full((1, 1)),
        full((_H, _H)), full((_R, _H, _H)), full((1, _H)),
        full((_H, _H)), full((_R, _H, _H)), full((1, _H)),
    ]

    out = pl.pallas_call(
        _rgcn_last_step_kernel,
        out_shape=jax.ShapeDtypeStruct((1, _N, 1), jnp.float32),
        grid=(1,),
        in_specs=in_specs,
        out_specs=pl.BlockSpec((1, _N, 1), lambda i: (0, 0, 0)),
        compiler_params=pltpu.CompilerParams(
            vmem_limit_bytes=100 * 1024 * 1024,
        ),
    )(
        x, adjs,
        fc1_w, fc1_b.reshape(1, _H), fc2_w, fc2_b.reshape(1, _H),
        fc3_w, fc3_b.reshape(1, _H),
        fc4_w.reshape(1, _H), fc4_b.reshape(1, 1),
        g0_wself, g0_wrel, g0_b.reshape(1, _H),
        g1_wself, g1_wrel, g1_b.reshape(1, _H),
    )
    return out
